# Initial kernel scaffold; baseline (speedup 1.0000x reference)
import jax
import jax.numpy as jnp
from jax.experimental import pallas as pl


def kernel(news_batch, word_table, ln_gamma, ln_beta):
    raise NotImplementedError("write your pallas kernel here")


# speedup vs baseline: 1.2667x; 1.2667x over previous
import functools

import jax
import jax.numpy as jnp
from jax import lax
from jax.experimental import pallas as pl
from jax.experimental.pallas import tpu as pltpu
from jax.experimental.pallas import tpu_sc as plsc

D = 768
LN_EPS = 1e-12
LANES = 16
NCH = D // LANES
G = 32


def _rsqrt16(v):
    i = lax.bitcast_convert_type(v, jnp.int32)
    y = lax.bitcast_convert_type(jnp.int32(0x5F3759DF) - (i >> 1), jnp.float32)
    for _ in range(3):
        y = y * (1.5 - 0.5 * v * y * y)
    return y


def _allsum16(x):
    lane = lax.iota(jnp.int32, LANES)
    for k in (8, 4, 2, 1):
        x = x + x.at[lane ^ k].get(mode="promise_in_bounds")
    return x


CB = 8


def _ln_group(rows_ref, out_ref, gamma_ref, beta_ref, mv_ref, rv_ref):
    def stats_body(i, carry):
        t0 = 2 * i
        t1 = t0 + 1
        z = jnp.zeros((LANES,), jnp.float32)
        s0a = s0b = q0a = q0b = z
        s1a = s1b = q1a = q1b = z
        for c in range(0, NCH, 2):
            xa0 = rows_ref[t0, pl.ds(c * LANES, LANES)]
            xb0 = rows_ref[t0, pl.ds((c + 1) * LANES, LANES)]
            xa1 = rows_ref[t1, pl.ds(c * LANES, LANES)]
            xb1 = rows_ref[t1, pl.ds((c + 1) * LANES, LANES)]
            s0a = s0a + xa0
            q0a = q0a + xa0 * xa0
            s0b = s0b + xb0
            q0b = q0b + xb0 * xb0
            s1a = s1a + xa1
            q1a = q1a + xa1 * xa1
            s1b = s1b + xb1
            q1b = q1b + xb1 * xb1
        m0 = _allsum16(s0a + s0b) * (1.0 / D)
        m1 = _allsum16(s1a + s1b) * (1.0 / D)
        v0 = _allsum16(q0a + q0b) * (1.0 / D) - m0 * m0
        v1 = _allsum16(q1a + q1b) * (1.0 / D) - m1 * m1
        r0 = _rsqrt16(v0 + LN_EPS)
        r1 = _rsqrt16(v1 + LN_EPS)
        mv_ref[t0, :] = m0
        rv_ref[t0, :] = r0
        mv_ref[t1, :] = m1
        rv_ref[t1, :] = r1
        return carry

    lax.fori_loop(0, G // 2, stats_body, 0)

    for cb in range(NCH // CB):
        gs = [gamma_ref[pl.ds((cb * CB + j) * LANES, LANES)] for j in range(CB)]
        bs = [beta_ref[pl.ds((cb * CB + j) * LANES, LANES)] for j in range(CB)]

        def norm_body(t, carry, cb=cb, gs=gs, bs=bs):
            mv = mv_ref[t, :]
            rv = rv_ref[t, :]
            for j in range(CB):
                sl = pl.ds((cb * CB + j) * LANES, LANES)
                x = rows_ref[t, sl]
                out_ref[t, sl] = (x - mv) * rv * gs[j] + bs[j]
            return carry

        lax.fori_loop(0, G, norm_body, 0)


def _build(n_tok):
    info = plsc.get_sparse_core_info()
    nc, ns = info.num_cores, info.num_subcores
    nw = nc * ns
    per_w = n_tok // nw
    ng = per_w // G
    assert per_w % G == 0 and ng % 2 == 0

    mesh = plsc.VectorSubcoreMesh(core_axis_name="c", subcore_axis_name="s")

    @functools.partial(
        pl.kernel,
        mesh=mesh,
        out_type=jax.ShapeDtypeStruct((n_tok, D), jnp.float32),
        scratch_types=[
            pltpu.VMEM((per_w,), jnp.int32),
            pltpu.VMEM((G, D), jnp.float32),
            pltpu.VMEM((G, D), jnp.float32),
            pltpu.VMEM((G, D), jnp.float32),
            pltpu.VMEM((G, D), jnp.float32),
            pltpu.VMEM((D,), jnp.float32),
            pltpu.VMEM((D,), jnp.float32),
            pltpu.VMEM((G, LANES), jnp.float32),
            pltpu.VMEM((G, LANES), jnp.float32),
            pltpu.SemaphoreType.DMA,
            pltpu.SemaphoreType.DMA,
            pltpu.SemaphoreType.DMA,
            pltpu.SemaphoreType.DMA,
        ],
    )
    def sc_embed_ln(table_hbm, ids_hbm, gamma_hbm, beta_hbm, out_hbm,
                    idx_v, rows0, rows1, ob0, ob1, gamma_v, beta_v,
                    mv_v, rv_v, si0, si1, so0, so1):
        rows = (rows0, rows1)
        obs = (ob0, ob1)
        sin = (si0, si1)
        sout = (so0, so1)

        wid = lax.axis_index("s") * nc + lax.axis_index("c")
        base = pl.multiple_of(wid * per_w, 8)
        pltpu.sync_copy(ids_hbm.at[pl.ds(base, per_w)], idx_v)
        pltpu.sync_copy(gamma_hbm, gamma_v)
        pltpu.sync_copy(beta_hbm, beta_v)

        def idx_slice(g):
            return idx_v.at[pl.ds(pl.multiple_of(g * G, 8), G)]

        def in_copy(g, b):
            return pltpu.make_async_copy(table_hbm.at[idx_slice(g)],
                                         rows[b], sin[b])

        def out_copy(g, b):
            return pltpu.make_async_copy(
                obs[b], out_hbm.at[pl.ds(base + g * G, G)], sout[b])

        in_copy(0, 0).start()
        in_copy(1, 1).start()

        def pair_body(i, carry):
            for b in (0, 1):
                g = 2 * i + b
                in_copy(g, b).wait()

                @pl.when(g >= 2)
                def _():
                    out_copy(g - 2, b).wait()

                _ln_group(rows[b], obs[b], gamma_v, beta_v, mv_v, rv_v)
                out_copy(g, b).start()

                @pl.when(g + 2 < ng)
                def _():
                    in_copy(g + 2, b).start()
            return carry

        lax.fori_loop(0, ng // 2, pair_body, 0)
        out_copy(ng - 2, 0).wait()
        out_copy(ng - 1, 1).wait()

    return sc_embed_ln


def kernel(news_batch, word_table, ln_gamma, ln_beta):
    b, l = news_batch.shape
    n_tok = b * l
    ids = news_batch.reshape(n_tok).astype(jnp.int32)
    out = _build(n_tok)(word_table, ids, ln_gamma, ln_beta)
    return out.reshape(b, l, D)
